# baseline (speedup 1.0000x reference)
import functools

import jax
import jax.numpy as jnp
from jax import lax
from jax.experimental import pallas as pl
from jax.experimental.pallas import tpu as pltpu
from jax.experimental.pallas import tpu_sc as plsc

N = 10000
E = 320000
D_NODE = 128
D_EDGE = 16
HID = 128
LAT = 64
MAXN = 50

NC = 2
NS = 16
L = 16
NW = NC * NS

E_REAL = E + N
K = 64
C = 162
RB = 3
RI = 6
EPW = C * K
EP = NW * EPW
RPT = 624

BLK = 2000
BE = 16000

_SDS = jax.ShapeDtypeStruct


def _node_pre_body(x_ref, w_ref, wsd_ref, xw_ref, a_ref, amax_ref):
    xw = jnp.dot(x_ref[...], w_ref[...], preferred_element_type=jnp.float32)
    xw_ref[...] = xw
    a = jnp.dot(xw, wsd_ref[...], preferred_element_type=jnp.float32)
    a_ref[...] = a

    @pl.when(pl.program_id(0) == 0)
    def _():
        amax_ref[0, 0] = -jnp.inf

    amax_ref[0, 0] = jnp.maximum(amax_ref[0, 0], jnp.max(a[:, 0]))


def _node_pre(x, w, wsd):
    return pl.pallas_call(
        _node_pre_body,
        grid=(N // BLK,),
        in_specs=[
            pl.BlockSpec((BLK, HID), lambda i: (i, 0)),
            pl.BlockSpec((HID, HID), lambda i: (0, 0)),
            pl.BlockSpec((HID, 8), lambda i: (0, 0)),
        ],
        out_specs=[
            pl.BlockSpec((BLK, HID), lambda i: (i, 0)),
            pl.BlockSpec((BLK, 8), lambda i: (i, 0)),
            pl.BlockSpec((1, 1), lambda i: (0, 0), memory_space=pltpu.SMEM),
        ],
        out_shape=[
            _SDS((N, HID), jnp.float32),
            _SDS((N, 8), jnp.float32),
            _SDS((1, 1), jnp.float32),
        ],
    )(x, w, wsd)


def _combine_pre_body(s0_ref, s1_ref, d_ref, b_ref, w_ref, wsd_ref,
                      xw_ref, a_ref, amax_ref):
    s = s0_ref[...] + s1_ref[...]
    ones = jnp.ones((NW, 1), jnp.float32)
    den = lax.dot_general(d_ref[0], ones, (((0,), (0,)), ((), ())),
                          preferred_element_type=jnp.float32)
    x = jnp.maximum(s / (den + 1e-16) + b_ref[...], 0.0)
    xw = jnp.dot(x, w_ref[...], preferred_element_type=jnp.float32)
    xw_ref[...] = xw
    a = jnp.dot(xw, wsd_ref[...], preferred_element_type=jnp.float32)
    a_ref[...] = a

    @pl.when(pl.program_id(0) == 0)
    def _():
        amax_ref[0, 0] = -jnp.inf

    amax_ref[0, 0] = jnp.maximum(amax_ref[0, 0], jnp.max(a[:, 0]))


def _combine_pre(s0, s1, d, b, w, wsd):
    return pl.pallas_call(
        _combine_pre_body,
        grid=(N // BLK,),
        in_specs=[
            pl.BlockSpec((BLK, HID), lambda i: (i, 0)),
            pl.BlockSpec((BLK, HID), lambda i: (i, 0)),
            pl.BlockSpec((1, NW, BLK), lambda i: (i, 0, 0)),
            pl.BlockSpec((1, HID), lambda i: (0, 0)),
            pl.BlockSpec((HID, HID), lambda i: (0, 0)),
            pl.BlockSpec((HID, 8), lambda i: (0, 0)),
        ],
        out_specs=[
            pl.BlockSpec((BLK, HID), lambda i: (i, 0)),
            pl.BlockSpec((BLK, 8), lambda i: (i, 0)),
            pl.BlockSpec((1, 1), lambda i: (0, 0), memory_space=pltpu.SMEM),
        ],
        out_shape=[
            _SDS((N, HID), jnp.float32),
            _SDS((N, 8), jnp.float32),
            _SDS((1, 1), jnp.float32),
        ],
    )(s0, s1, d, b, w, wsd)


def _final_mean_body(s0_ref, s1_ref, d_ref, b_ref, xsum_ref):
    s = s0_ref[...] + s1_ref[...]
    ones = jnp.ones((NW, 1), jnp.float32)
    den = lax.dot_general(d_ref[0], ones, (((0,), (0,)), ((), ())),
                          preferred_element_type=jnp.float32)
    x = jnp.maximum(s / (den + 1e-16) + b_ref[...], 0.0)

    @pl.when(pl.program_id(0) == 0)
    def _():
        xsum_ref[...] = jnp.zeros_like(xsum_ref)

    xsum_ref[...] += jnp.sum(x, axis=0, keepdims=True)


def _final_mean(s0, s1, d, b):
    return pl.pallas_call(
        _final_mean_body,
        grid=(N // BLK,),
        in_specs=[
            pl.BlockSpec((BLK, HID), lambda i: (i, 0)),
            pl.BlockSpec((BLK, HID), lambda i: (i, 0)),
            pl.BlockSpec((1, NW, BLK), lambda i: (i, 0, 0)),
            pl.BlockSpec((1, HID), lambda i: (0, 0)),
        ],
        out_specs=pl.BlockSpec((1, HID), lambda i: (0, 0)),
        out_shape=_SDS((1, HID), jnp.float32),
    )(s0, s1, d, b)


def _edge_mean_body(ef_ref, we_ref, be_ref, esum_ref):
    h = jnp.maximum(
        jnp.dot(ef_ref[...], we_ref[...], preferred_element_type=jnp.float32)
        + be_ref[...], 0.0)

    @pl.when(pl.program_id(0) == 0)
    def _():
        esum_ref[...] = jnp.zeros_like(esum_ref)

    esum_ref[...] += jnp.sum(h, axis=0, keepdims=True)


def _edge_mean(ef, we, be):
    return pl.pallas_call(
        _edge_mean_body,
        grid=(E // BE,),
        in_specs=[
            pl.BlockSpec((BE, D_EDGE), lambda i: (i, 0)),
            pl.BlockSpec((D_EDGE, HID), lambda i: (0, 0)),
            pl.BlockSpec((1, HID), lambda i: (0, 0)),
        ],
        out_specs=pl.BlockSpec((1, HID), lambda i: (0, 0)),
        out_shape=_SDS((1, HID), jnp.float32),
    )(ef, we, be)


def _decoder_body(ge_ref, eps_ref, wc_ref, bc_ref, wmu_ref, bmu_ref,
                  wlv_ref, blv_ref, wl2h_ref, bl2h_ref, wnh_ref, bnh_ref,
                  wnf_ref, bnf_ref, weh_ref, beh_ref, wee_ref, bee_ref,
                  wnn_ref, bnn_ref, wcp_ref, bcp_ref,
                  nf_ref, ee_ref, nn_ref, cp_ref, mu_ref, lv_ref):
    dot = lambda a, b: jnp.dot(a, b, preferred_element_type=jnp.float32)
    ge = jnp.maximum(dot(ge_ref[...], wc_ref[...]) + bc_ref[...], 0.0)
    mu = dot(ge, wmu_ref[...]) + bmu_ref[...]
    lv = dot(ge, wlv_ref[...]) + blv_ref[...]
    mu_ref[...] = mu
    lv_ref[...] = lv
    z = mu + eps_ref[...] * jnp.exp(0.5 * lv)
    h = jnp.maximum(dot(z, wl2h_ref[...]) + bl2h_ref[...], 0.0)
    nh = jnp.maximum(dot(h, wnh_ref[...]) + bnh_ref[...], 0.0)
    nf_ref[...] = dot(nh, wnf_ref[...]) + bnf_ref[...]
    eh = jnp.maximum(dot(h, weh_ref[...]) + beh_ref[...], 0.0)
    ee_ref[...] = dot(eh, wee_ref[...]) + bee_ref[...]
    nn_ref[...] = dot(h, wnn_ref[...]) + bnn_ref[...]
    cp_ref[...] = dot(h, wcp_ref[...]) + bcp_ref[...]


def _decoder(ge, eps, wc, bc, wmu, bmu, wlv, blv, wl2h, bl2h, wnh, bnh,
             wnf, bnf, weh, beh, wee, bee, wnn, bnn, wcp, bcp):
    return pl.pallas_call(
        _decoder_body,
        out_shape=[
            _SDS((1, D_NODE * MAXN), jnp.float32),
            _SDS((1, MAXN * MAXN), jnp.float32),
            _SDS((1, MAXN), jnp.float32),
            _SDS((1, 6), jnp.float32),
            _SDS((1, LAT), jnp.float32),
            _SDS((1, LAT), jnp.float32),
        ],
    )(ge, eps, wc, bc, wmu, bmu, wlv, blv, wl2h, bl2h, wnh, bnh,
      wnf, bnf, weh, beh, wee, bee, wnn, bnn, wcp, bcp)


def _sc_gat_body(xw_hbm, asrc_hbm, adst_hbm, amax_hbm, idx_hbm,
                 s_out, d_out,
                 amax_v, idxb_v, asb_v, adb_v, rows_v, ex_v, den_v,
                 asrc_sh, adst_sh, s_sh,
                 semr0, semr1, semr2, semc0, semc1, semc2,
                 semw0, semw1, semw2, semi0, semi1, semi2, semi3, semi4, semi5):
    cid = lax.axis_index("c")
    sid = lax.axis_index("s")
    wid = cid * NS + sid
    semr = (semr0, semr1, semr2)
    semc = (semc0, semc1, semc2)
    semw = (semw0, semw1, semw2)
    semi = (semi0, semi1, semi2, semi3, semi4, semi5)

    pltpu.sync_copy(amax_hbm, amax_v)

    @pl.when(sid == 0)
    def _():
        pltpu.sync_copy(asrc_hbm, asrc_sh)
        pltpu.sync_copy(adst_hbm, adst_sh)

    plsc.subcore_barrier()

    zero = jnp.zeros((L,), jnp.float32)

    def _zero_row(r, carry):
        for c in range(HID // L):
            rows_v[0, r, pl.ds(c * L, L)] = zero
        return carry

    lax.fori_loop(0, K, _zero_row, 0)

    def _zero_den(i, carry):
        den_v[pl.ds(i * L, L)] = zero
        return carry

    lax.fori_loop(0, N // L, _zero_den, 0)

    av = amax_v[...]
    liota = lax.iota(jnp.int32, L)

    pltpu.sync_copy(idx_hbm.at[wid, 0], idxb_v.at[0])
    pltpu.sync_copy(idx_hbm.at[wid, 1], idxb_v.at[1])
    pltpu.async_copy(idx_hbm.at[wid, 2], idxb_v.at[2], semi[2])
    pltpu.async_copy(asrc_sh.at[idxb_v.at[0, 0]], asb_v.at[0], semc[0])
    pltpu.async_copy(adst_sh.at[idxb_v.at[0, 1]], adb_v.at[0], semc[0])
    pltpu.async_copy(xw_hbm.at[idxb_v.at[1, 0]], rows_v.at[1], semr[1])
    pltpu.async_copy(asrc_sh.at[idxb_v.at[1, 0]], asb_v.at[1], semc[1])
    pltpu.async_copy(adst_sh.at[idxb_v.at[1, 1]], adb_v.at[1], semc[1])

    zbase = sid * RPT
    for i in range(10):
        pltpu.sync_copy(rows_v.at[0], s_sh.at[pl.ds(zbase + i * K, K)])
    pltpu.async_copy(xw_hbm.at[idxb_v.at[0, 0]], rows_v.at[0], semr[0])
    plsc.subcore_barrier()

    def _outer(o, carry):
        for g in range(RI):
            ch = o * RI + g
            r = g % RB
            r2 = (g + 2) % RB
            g2 = (g + 2) % RI
            g3 = (g + 3) % RI

            @pl.when(ch + 2 < C)
            def _(ch=ch, r2=r2, g2=g2):
                pltpu.make_async_copy(idx_hbm.at[wid, 0], idxb_v.at[g2],
                                      semi[g2]).wait()

                @pl.when(ch >= 1)
                def _():
                    pltpu.make_async_copy(rows_v.at[r2], s_sh.at[pl.ds(0, K)],
                                          semw[r2]).wait()

                pltpu.async_copy(xw_hbm.at[idxb_v.at[g2, 0]], rows_v.at[r2],
                                 semr[r2])
                pltpu.async_copy(asrc_sh.at[idxb_v.at[g2, 0]], asb_v.at[r2],
                                 semc[r2])
                pltpu.async_copy(adst_sh.at[idxb_v.at[g2, 1]], adb_v.at[r2],
                                 semc[r2])

            @pl.when(ch + 3 < C)
            def _(ch=ch, g3=g3):
                pltpu.async_copy(idx_hbm.at[wid, ch + 3], idxb_v.at[g3],
                                 semi[g3])

            pltpu.make_async_copy(xw_hbm.at[pl.ds(0, K)], rows_v.at[r],
                                  semr[r]).wait()
            pltpu.make_async_copy(asrc_hbm.at[pl.ds(0, K)], asb_v.at[r],
                                  semc[r]).wait()
            pltpu.make_async_copy(asrc_hbm.at[pl.ds(0, K)], adb_v.at[r],
                                  semc[r]).wait()
            ebase = wid * EPW + ch * K

            def _escalar(j, carry2, r=r, g=g, ebase=ebase):
                asv = asb_v[r, pl.ds(j * L, L)]
                adv = adb_v[r, pl.ds(j * L, L)]
                didx = idxb_v[g, 1, pl.ds(j * L, L)]
                s = asv + adv
                e = jnp.where(s >= 0.0, s, 0.2 * s)
                t = av + adv
                cb = jnp.where(t >= 0.0, t, 0.2 * t)
                ex = jnp.exp(e - cb)
                eid = ebase + j * L + liota
                ex = jnp.where(eid < E_REAL, ex, 0.0)
                ex_v[pl.ds(j * L, L)] = ex
                for l in range(L):
                    plsc.addupdate_scatter(den_v, [didx], ex, mask=liota == l)
                return carry2

            lax.fori_loop(0, K // L, _escalar, 0)

            @plsc.parallel_loop(0, K, 1, unroll=8)
            def _scale_row(rr, r=r):
                bl = plsc.load_gather(ex_v, [jnp.full((L,), rr, jnp.int32)])
                for c in range(HID // L):
                    rows_v[r, rr, pl.ds(c * L, L)] = (
                        rows_v[r, rr, pl.ds(c * L, L)] * bl)

            pltpu.async_copy(rows_v.at[r], s_sh.at[idxb_v.at[g, 1]],
                             semw[r], add=True)
        return carry

    lax.fori_loop(0, C // RI, _outer, 0)
    for r in range(RB):
        pltpu.make_async_copy(rows_v.at[r], s_sh.at[pl.ds(0, K)],
                              semw[r]).wait()
    plsc.subcore_barrier()

    tail = N - NS * RPT
    pltpu.sync_copy(s_sh.at[pl.ds(sid * RPT, RPT)],
                    s_out.at[cid, pl.ds(sid * RPT, RPT)])

    @pl.when(sid == NS - 1)
    def _():
        pltpu.sync_copy(s_sh.at[pl.ds(NS * RPT, tail)],
                        s_out.at[cid, pl.ds(NS * RPT, tail)])

    pltpu.sync_copy(den_v, d_out.at[wid])


def _sc_gat(xw, asrc, adst, amaxv, idx):
    mesh = plsc.VectorSubcoreMesh(
        core_axis_name="c", subcore_axis_name="s",
        num_cores=NC, num_subcores=NS)
    f = pl.kernel(
        _sc_gat_body,
        out_type=(
            _SDS((NC, N, HID), jnp.float32),
            _SDS((NW, N), jnp.float32),
        ),
        mesh=mesh,
        compiler_params=pltpu.CompilerParams(needs_layout_passes=False),
        scratch_types=[
            pltpu.VMEM((L,), jnp.float32),
            pltpu.VMEM((RI, 2, K), jnp.int32),
            pltpu.VMEM((RB, K), jnp.float32),
            pltpu.VMEM((RB, K), jnp.float32),
            pltpu.VMEM((RB, K, HID), jnp.float32),
            pltpu.VMEM((K,), jnp.float32),
            pltpu.VMEM((N,), jnp.float32),
            pltpu.VMEM_SHARED((N,), jnp.float32),
            pltpu.VMEM_SHARED((N,), jnp.float32),
            pltpu.VMEM_SHARED((N, HID), jnp.float32),
        ] + [pltpu.SemaphoreType.DMA] * (3 * RB + RI),
    )
    return f(xw, asrc, adst, amaxv, idx)


def kernel(node_features, edge_index, edge_features, W1, b1, as1, ad1, W2, b2,
           as2, ad2, We, be, Wc, bc, Wmu, bmu, Wlv, blv, Wl2h, bl2h, Wnh, bnh,
           Wnf, bnf, Weh, beh, Wee, bee, Wnn, bnn, Wcp, bcp):
    loop = jnp.arange(N, dtype=edge_index.dtype)
    src = jnp.concatenate([edge_index[0], loop])
    dst = jnp.concatenate([edge_index[1], loop])
    pad = EP - E_REAL
    src = jnp.pad(src, (0, pad)).reshape(NW, C, K)
    dst = jnp.pad(dst, (0, pad)).reshape(NW, C, K)
    idx = jnp.stack([src, dst], axis=2)

    zpad = jnp.zeros((HID, 6), jnp.float32)
    wsd1 = jnp.concatenate([as1[:, None], ad1[:, None], zpad], axis=1)
    wsd2 = jnp.concatenate([as2[:, None], ad2[:, None], zpad], axis=1)

    xw1, a1, amax1 = _node_pre(node_features, W1, wsd1)
    amaxv1 = jnp.full((L,), amax1[0, 0], jnp.float32)
    S1, D1 = _sc_gat(xw1, a1[:, 0], a1[:, 1], amaxv1, idx)

    D1r = D1.reshape(NW, N // BLK, BLK).transpose(1, 0, 2)
    xw2, a2, amax2 = _combine_pre(S1[0], S1[1], D1r,
                                  b1.reshape(1, HID), W2, wsd2)
    amaxv2 = jnp.full((L,), amax2[0, 0], jnp.float32)
    S2, D2 = _sc_gat(xw2, a2[:, 0], a2[:, 1], amaxv2, idx)

    D2r = D2.reshape(NW, N // BLK, BLK).transpose(1, 0, 2)
    xsum = _final_mean(S2[0], S2[1], D2r, b2.reshape(1, HID))
    esum = _edge_mean(edge_features, We, be.reshape(1, HID))
    ge_in = jnp.concatenate([xsum * (1.0 / N), esum * (1.0 / E)], axis=1)

    eps = jax.random.normal(jax.random.key(42), (1, LAT), jnp.float32)
    nf, ee, nn, cp, mu, lv = _decoder(
        ge_in, eps, Wc, bc.reshape(1, -1), Wmu, bmu.reshape(1, -1),
        Wlv, blv.reshape(1, -1), Wl2h, bl2h.reshape(1, -1),
        Wnh, bnh.reshape(1, -1), Wnf, bnf.reshape(1, -1),
        Weh, beh.reshape(1, -1), Wee, bee.reshape(1, -1),
        Wnn, bnn.reshape(1, -1), Wcp, bcp.reshape(1, -1))

    return (nf.reshape(1, MAXN, D_NODE), ee.reshape(1, MAXN, MAXN),
            nn, cp, mu, lv)


# speedup vs baseline: 1.0124x; 1.0124x over previous
import functools

import jax
import jax.numpy as jnp
from jax import lax
from jax.experimental import pallas as pl
from jax.experimental.pallas import tpu as pltpu
from jax.experimental.pallas import tpu_sc as plsc

N = 10000
E = 320000
D_NODE = 128
D_EDGE = 16
HID = 128
LAT = 64
MAXN = 50

NC = 2
NS = 16
L = 16
NW = NC * NS

E_REAL = E + N
K = 64
C = 162
RB = 3
RI = 6
EPW = C * K
EP = NW * EPW
RPT = 624

BLK = 2000
BE = 16000

_SDS = jax.ShapeDtypeStruct


def _node_pre_body(x_ref, w_ref, wsd_ref, xw_ref, a_ref, amax_ref):
    xw = jnp.dot(x_ref[...], w_ref[...], preferred_element_type=jnp.float32)
    xw_ref[...] = xw
    a = jnp.dot(xw, wsd_ref[...], preferred_element_type=jnp.float32)
    a_ref[...] = a

    @pl.when(pl.program_id(0) == 0)
    def _():
        amax_ref[0, 0] = -jnp.inf

    amax_ref[0, 0] = jnp.maximum(amax_ref[0, 0], jnp.max(a[:, 0]))


def _node_pre(x, w, wsd):
    return pl.pallas_call(
        _node_pre_body,
        grid=(N // BLK,),
        in_specs=[
            pl.BlockSpec((BLK, HID), lambda i: (i, 0)),
            pl.BlockSpec((HID, HID), lambda i: (0, 0)),
            pl.BlockSpec((HID, 8), lambda i: (0, 0)),
        ],
        out_specs=[
            pl.BlockSpec((BLK, HID), lambda i: (i, 0)),
            pl.BlockSpec((BLK, 8), lambda i: (i, 0)),
            pl.BlockSpec((1, 1), lambda i: (0, 0), memory_space=pltpu.SMEM),
        ],
        out_shape=[
            _SDS((N, HID), jnp.float32),
            _SDS((N, 8), jnp.float32),
            _SDS((1, 1), jnp.float32),
        ],
    )(x, w, wsd)


def _combine_pre_body(s0_ref, s1_ref, d_ref, b_ref, w_ref, wsd_ref,
                      xw_ref, a_ref, amax_ref):
    s = s0_ref[...] + s1_ref[...]
    ones = jnp.ones((NW, 1), jnp.float32)
    den = lax.dot_general(d_ref[0], ones, (((0,), (0,)), ((), ())),
                          preferred_element_type=jnp.float32)
    x = jnp.maximum(s / (den + 1e-16) + b_ref[...], 0.0)
    xw = jnp.dot(x, w_ref[...], preferred_element_type=jnp.float32)
    xw_ref[...] = xw
    a = jnp.dot(xw, wsd_ref[...], preferred_element_type=jnp.float32)
    a_ref[...] = a

    @pl.when(pl.program_id(0) == 0)
    def _():
        amax_ref[0, 0] = -jnp.inf

    amax_ref[0, 0] = jnp.maximum(amax_ref[0, 0], jnp.max(a[:, 0]))


def _combine_pre(s0, s1, d, b, w, wsd):
    return pl.pallas_call(
        _combine_pre_body,
        grid=(N // BLK,),
        in_specs=[
            pl.BlockSpec((BLK, HID), lambda i: (i, 0)),
            pl.BlockSpec((BLK, HID), lambda i: (i, 0)),
            pl.BlockSpec((1, NW, BLK), lambda i: (i, 0, 0)),
            pl.BlockSpec((1, HID), lambda i: (0, 0)),
            pl.BlockSpec((HID, HID), lambda i: (0, 0)),
            pl.BlockSpec((HID, 8), lambda i: (0, 0)),
        ],
        out_specs=[
            pl.BlockSpec((BLK, HID), lambda i: (i, 0)),
            pl.BlockSpec((BLK, 8), lambda i: (i, 0)),
            pl.BlockSpec((1, 1), lambda i: (0, 0), memory_space=pltpu.SMEM),
        ],
        out_shape=[
            _SDS((N, HID), jnp.float32),
            _SDS((N, 8), jnp.float32),
            _SDS((1, 1), jnp.float32),
        ],
    )(s0, s1, d, b, w, wsd)


def _final_mean_body(s0_ref, s1_ref, d_ref, b_ref, xsum_ref):
    s = s0_ref[...] + s1_ref[...]
    ones = jnp.ones((NW, 1), jnp.float32)
    den = lax.dot_general(d_ref[0], ones, (((0,), (0,)), ((), ())),
                          preferred_element_type=jnp.float32)
    x = jnp.maximum(s / (den + 1e-16) + b_ref[...], 0.0)

    @pl.when(pl.program_id(0) == 0)
    def _():
        xsum_ref[...] = jnp.zeros_like(xsum_ref)

    xsum_ref[...] += jnp.sum(x, axis=0, keepdims=True)


def _final_mean(s0, s1, d, b):
    return pl.pallas_call(
        _final_mean_body,
        grid=(N // BLK,),
        in_specs=[
            pl.BlockSpec((BLK, HID), lambda i: (i, 0)),
            pl.BlockSpec((BLK, HID), lambda i: (i, 0)),
            pl.BlockSpec((1, NW, BLK), lambda i: (i, 0, 0)),
            pl.BlockSpec((1, HID), lambda i: (0, 0)),
        ],
        out_specs=pl.BlockSpec((1, HID), lambda i: (0, 0)),
        out_shape=_SDS((1, HID), jnp.float32),
    )(s0, s1, d, b)


def _edge_mean_body(ef_ref, we_ref, be_ref, esum_ref):
    h = jnp.maximum(
        jnp.dot(ef_ref[...], we_ref[...], preferred_element_type=jnp.float32)
        + be_ref[...], 0.0)

    @pl.when(pl.program_id(0) == 0)
    def _():
        esum_ref[...] = jnp.zeros_like(esum_ref)

    esum_ref[...] += jnp.sum(h, axis=0, keepdims=True)


def _edge_mean(ef, we, be):
    return pl.pallas_call(
        _edge_mean_body,
        grid=(E // BE,),
        in_specs=[
            pl.BlockSpec((BE, D_EDGE), lambda i: (i, 0)),
            pl.BlockSpec((D_EDGE, HID), lambda i: (0, 0)),
            pl.BlockSpec((1, HID), lambda i: (0, 0)),
        ],
        out_specs=pl.BlockSpec((1, HID), lambda i: (0, 0)),
        out_shape=_SDS((1, HID), jnp.float32),
    )(ef, we, be)


def _decoder_body(ge_ref, eps_ref, wc_ref, bc_ref, wmu_ref, bmu_ref,
                  wlv_ref, blv_ref, wl2h_ref, bl2h_ref, wnh_ref, bnh_ref,
                  wnf_ref, bnf_ref, weh_ref, beh_ref, wee_ref, bee_ref,
                  wnn_ref, bnn_ref, wcp_ref, bcp_ref,
                  nf_ref, ee_ref, nn_ref, cp_ref, mu_ref, lv_ref):
    dot = lambda a, b: jnp.dot(a, b, preferred_element_type=jnp.float32)
    ge = jnp.maximum(dot(ge_ref[...], wc_ref[...]) + bc_ref[...], 0.0)
    mu = dot(ge, wmu_ref[...]) + bmu_ref[...]
    lv = dot(ge, wlv_ref[...]) + blv_ref[...]
    mu_ref[...] = mu
    lv_ref[...] = lv
    z = mu + eps_ref[...] * jnp.exp(0.5 * lv)
    h = jnp.maximum(dot(z, wl2h_ref[...]) + bl2h_ref[...], 0.0)
    nh = jnp.maximum(dot(h, wnh_ref[...]) + bnh_ref[...], 0.0)
    nf_ref[...] = dot(nh, wnf_ref[...]) + bnf_ref[...]
    eh = jnp.maximum(dot(h, weh_ref[...]) + beh_ref[...], 0.0)
    ee_ref[...] = dot(eh, wee_ref[...]) + bee_ref[...]
    nn_ref[...] = dot(h, wnn_ref[...]) + bnn_ref[...]
    cp_ref[...] = dot(h, wcp_ref[...]) + bcp_ref[...]


def _decoder(ge, eps, wc, bc, wmu, bmu, wlv, blv, wl2h, bl2h, wnh, bnh,
             wnf, bnf, weh, beh, wee, bee, wnn, bnn, wcp, bcp):
    return pl.pallas_call(
        _decoder_body,
        out_shape=[
            _SDS((1, D_NODE * MAXN), jnp.float32),
            _SDS((1, MAXN * MAXN), jnp.float32),
            _SDS((1, MAXN), jnp.float32),
            _SDS((1, 6), jnp.float32),
            _SDS((1, LAT), jnp.float32),
            _SDS((1, LAT), jnp.float32),
        ],
    )(ge, eps, wc, bc, wmu, bmu, wlv, blv, wl2h, bl2h, wnh, bnh,
      wnf, bnf, weh, beh, wee, bee, wnn, bnn, wcp, bcp)


def _sc_gat_body(xw_hbm, asrc_hbm, adst_hbm, amax_hbm, idx_hbm,
                 s_out, d_out,
                 amax_v, idxb_v, asb_v, adb_v, rows_v, ex_v, den_v,
                 asrc_sh, adst_sh, s_sh,
                 semr0, semr1, semr2, semc0, semc1, semc2,
                 semw0, semw1, semw2, semi0, semi1, semi2, semi3, semi4, semi5):
    cid = lax.axis_index("c")
    sid = lax.axis_index("s")
    wid = cid * NS + sid
    semr = (semr0, semr1, semr2)
    semc = (semc0, semc1, semc2)
    semw = (semw0, semw1, semw2)
    semi = (semi0, semi1, semi2, semi3, semi4, semi5)

    pltpu.sync_copy(amax_hbm, amax_v)

    @pl.when(sid == 0)
    def _():
        pltpu.sync_copy(asrc_hbm, asrc_sh)
        pltpu.sync_copy(adst_hbm, adst_sh)

    zero = jnp.zeros((L,), jnp.float32)

    def _zero_row(r, carry):
        for c in range(HID // L):
            rows_v[0, r, pl.ds(c * L, L)] = zero
        return carry

    lax.fori_loop(0, K, _zero_row, 0)

    def _zero_den(i, carry):
        den_v[pl.ds(i * L, L)] = zero
        return carry

    lax.fori_loop(0, N // L, _zero_den, 0)

    zbase = sid * RPT
    for i in range(10):
        pltpu.sync_copy(rows_v.at[0], s_sh.at[pl.ds(zbase + i * K, K)])
    plsc.subcore_barrier()

    av = amax_v[...]
    liota = lax.iota(jnp.int32, L)

    pltpu.sync_copy(idx_hbm.at[wid, 0], idxb_v.at[0])
    pltpu.sync_copy(idx_hbm.at[wid, 1], idxb_v.at[1])
    pltpu.async_copy(idx_hbm.at[wid, 2], idxb_v.at[2], semi[2])
    for b in range(2):
        pltpu.async_copy(xw_hbm.at[idxb_v.at[b, 0]], rows_v.at[b], semr[b])
        pltpu.async_copy(asrc_sh.at[idxb_v.at[b, 0]], asb_v.at[b], semc[b])
        pltpu.async_copy(adst_sh.at[idxb_v.at[b, 1]], adb_v.at[b], semc[b])

    def _outer(o, carry):
        for g in range(RI):
            ch = o * RI + g
            r = g % RB
            r2 = (g + 2) % RB
            g2 = (g + 2) % RI
            g3 = (g + 3) % RI

            @pl.when(ch + 2 < C)
            def _(ch=ch, r2=r2, g2=g2):
                pltpu.make_async_copy(idx_hbm.at[wid, 0], idxb_v.at[g2],
                                      semi[g2]).wait()

                @pl.when(ch >= 1)
                def _():
                    pltpu.make_async_copy(rows_v.at[r2], s_sh.at[pl.ds(0, K)],
                                          semw[r2]).wait()

                pltpu.async_copy(xw_hbm.at[idxb_v.at[g2, 0]], rows_v.at[r2],
                                 semr[r2])
                pltpu.async_copy(asrc_sh.at[idxb_v.at[g2, 0]], asb_v.at[r2],
                                 semc[r2])
                pltpu.async_copy(adst_sh.at[idxb_v.at[g2, 1]], adb_v.at[r2],
                                 semc[r2])

            @pl.when(ch + 3 < C)
            def _(ch=ch, g3=g3):
                pltpu.async_copy(idx_hbm.at[wid, ch + 3], idxb_v.at[g3],
                                 semi[g3])

            pltpu.make_async_copy(xw_hbm.at[pl.ds(0, K)], rows_v.at[r],
                                  semr[r]).wait()
            pltpu.make_async_copy(asrc_hbm.at[pl.ds(0, K)], asb_v.at[r],
                                  semc[r]).wait()
            pltpu.make_async_copy(asrc_hbm.at[pl.ds(0, K)], adb_v.at[r],
                                  semc[r]).wait()
            ebase = wid * EPW + ch * K

            def _escalar(j, carry2, r=r, g=g, ebase=ebase):
                asv = asb_v[r, pl.ds(j * L, L)]
                adv = adb_v[r, pl.ds(j * L, L)]
                didx = idxb_v[g, 1, pl.ds(j * L, L)]
                s = asv + adv
                e = jnp.where(s >= 0.0, s, 0.2 * s)
                t = av + adv
                cb = jnp.where(t >= 0.0, t, 0.2 * t)
                ex = jnp.exp(e - cb)
                eid = ebase + j * L + liota
                ex = jnp.where(eid < E_REAL, ex, 0.0)
                ex_v[pl.ds(j * L, L)] = ex
                for l in range(L):
                    plsc.addupdate_scatter(den_v, [didx], ex, mask=liota == l)
                return carry2

            lax.fori_loop(0, K // L, _escalar, 0)

            @plsc.parallel_loop(0, K, 1, unroll=4)
            def _scale_row(rr, r=r):
                bl = plsc.load_gather(ex_v, [jnp.full((L,), rr, jnp.int32)])
                for c in range(HID // L):
                    rows_v[r, rr, pl.ds(c * L, L)] = (
                        rows_v[r, rr, pl.ds(c * L, L)] * bl)

            pltpu.async_copy(rows_v.at[r], s_sh.at[idxb_v.at[g, 1]],
                             semw[r], add=True)
        return carry

    lax.fori_loop(0, C // RI, _outer, 0)
    for r in range(RB):
        pltpu.make_async_copy(rows_v.at[r], s_sh.at[pl.ds(0, K)],
                              semw[r]).wait()
    plsc.subcore_barrier()

    tail = N - NS * RPT
    pltpu.sync_copy(s_sh.at[pl.ds(sid * RPT, RPT)],
                    s_out.at[cid, pl.ds(sid * RPT, RPT)])

    @pl.when(sid == NS - 1)
    def _():
        pltpu.sync_copy(s_sh.at[pl.ds(NS * RPT, tail)],
                        s_out.at[cid, pl.ds(NS * RPT, tail)])

    pltpu.sync_copy(den_v, d_out.at[wid])


def _sc_gat(xw, asrc, adst, amaxv, idx):
    mesh = plsc.VectorSubcoreMesh(
        core_axis_name="c", subcore_axis_name="s",
        num_cores=NC, num_subcores=NS)
    f = pl.kernel(
        _sc_gat_body,
        out_type=(
            _SDS((NC, N, HID), jnp.float32),
            _SDS((NW, N), jnp.float32),
        ),
        mesh=mesh,
        compiler_params=pltpu.CompilerParams(needs_layout_passes=False),
        scratch_types=[
            pltpu.VMEM((L,), jnp.float32),
            pltpu.VMEM((RI, 2, K), jnp.int32),
            pltpu.VMEM((RB, K), jnp.float32),
            pltpu.VMEM((RB, K), jnp.float32),
            pltpu.VMEM((RB, K, HID), jnp.float32),
            pltpu.VMEM((K,), jnp.float32),
            pltpu.VMEM((N,), jnp.float32),
            pltpu.VMEM_SHARED((N,), jnp.float32),
            pltpu.VMEM_SHARED((N,), jnp.float32),
            pltpu.VMEM_SHARED((N, HID), jnp.float32),
        ] + [pltpu.SemaphoreType.DMA] * (3 * RB + RI),
    )
    return f(xw, asrc, adst, amaxv, idx)


def kernel(node_features, edge_index, edge_features, W1, b1, as1, ad1, W2, b2,
           as2, ad2, We, be, Wc, bc, Wmu, bmu, Wlv, blv, Wl2h, bl2h, Wnh, bnh,
           Wnf, bnf, Weh, beh, Wee, bee, Wnn, bnn, Wcp, bcp):
    loop = jnp.arange(N, dtype=edge_index.dtype)
    src = jnp.concatenate([edge_index[0], loop])
    dst = jnp.concatenate([edge_index[1], loop])
    pad = EP - E_REAL
    src = jnp.pad(src, (0, pad)).reshape(NW, C, K)
    dst = jnp.pad(dst, (0, pad)).reshape(NW, C, K)
    idx = jnp.stack([src, dst], axis=2)

    zpad = jnp.zeros((HID, 6), jnp.float32)
    wsd1 = jnp.concatenate([as1[:, None], ad1[:, None], zpad], axis=1)
    wsd2 = jnp.concatenate([as2[:, None], ad2[:, None], zpad], axis=1)

    xw1, a1, amax1 = _node_pre(node_features, W1, wsd1)
    amaxv1 = jnp.full((L,), amax1[0, 0], jnp.float32)
    S1, D1 = _sc_gat(xw1, a1[:, 0], a1[:, 1], amaxv1, idx)

    D1r = D1.reshape(NW, N // BLK, BLK).transpose(1, 0, 2)
    xw2, a2, amax2 = _combine_pre(S1[0], S1[1], D1r,
                                  b1.reshape(1, HID), W2, wsd2)
    amaxv2 = jnp.full((L,), amax2[0, 0], jnp.float32)
    S2, D2 = _sc_gat(xw2, a2[:, 0], a2[:, 1], amaxv2, idx)

    D2r = D2.reshape(NW, N // BLK, BLK).transpose(1, 0, 2)
    xsum = _final_mean(S2[0], S2[1], D2r, b2.reshape(1, HID))
    esum = _edge_mean(edge_features, We, be.reshape(1, HID))
    ge_in = jnp.concatenate([xsum * (1.0 / N), esum * (1.0 / E)], axis=1)

    eps = jax.random.normal(jax.random.key(42), (1, LAT), jnp.float32)
    nf, ee, nn, cp, mu, lv = _decoder(
        ge_in, eps, Wc, bc.reshape(1, -1), Wmu, bmu.reshape(1, -1),
        Wlv, blv.reshape(1, -1), Wl2h, bl2h.reshape(1, -1),
        Wnh, bnh.reshape(1, -1), Wnf, bnf.reshape(1, -1),
        Weh, beh.reshape(1, -1), Wee, bee.reshape(1, -1),
        Wnn, bnn.reshape(1, -1), Wcp, bcp.reshape(1, -1))

    return (nf.reshape(1, MAXN, D_NODE), ee.reshape(1, MAXN, MAXN),
            nn, cp, mu, lv)
